# baseline (speedup 1.0000x reference)
import jax
import jax.numpy as jnp
from jax.experimental import pallas as pl
from jax.experimental.pallas import tpu as pltpu

_GROUP = 2
_NCHUNK = 16 // _GROUP
_NBUF = _NCHUNK
_RAHEAD = 7


def _trim_kernel(x_hbm, v_hbm, m_hbm, xo_hbm, vo_hbm, mo_ref,
                 xbuf, vbuf, mbuf, rsem, wsem, vsem, msem):
    def src(ref, i):
        return ref.at[pl.ds(i * _GROUP, _GROUP)]

    def rd(i):
        return pltpu.make_async_copy(
            src(x_hbm, i), xbuf.at[i % _NBUF], rsem.at[i % _NBUF])

    def wr(i):
        return pltpu.make_async_copy(
            xbuf.at[i % _NBUF], src(xo_hbm, i), wsem.at[i % _NBUF])

    for i in range(_RAHEAD):
        rd(i).start()

    m_rd = pltpu.make_async_copy(m_hbm, mbuf, msem.at[0])
    m_rd.start()
    v_rd = pltpu.make_async_copy(v_hbm, vbuf, vsem.at[0])
    v_wr = pltpu.make_async_copy(vbuf, vo_hbm, vsem.at[1])
    v_rd.start()
    m_rd.wait()
    mo_ref[...] = mbuf[...] != 0.0
    v_rd.wait()
    v_wr.start()

    for i in range(_NCHUNK):
        rd(i).wait()
        wr(i).start()
        nxt = i + _RAHEAD
        if nxt < _NCHUNK:
            if nxt >= _NBUF:
                wr(nxt - _NBUF).wait()
            rd(nxt).start()
    for i in range(_NCHUNK - min(_NBUF, _NCHUNK), _NCHUNK):
        wr(i).wait()
    v_wr.wait()


def _trim(x, v, mask):
    hbm = pl.BlockSpec(memory_space=pltpu.MemorySpace.HBM)
    return pl.pallas_call(
        _trim_kernel,
        in_specs=[hbm, hbm, hbm],
        out_specs=[hbm, hbm,
                   pl.BlockSpec(memory_space=pltpu.MemorySpace.VMEM)],
        out_shape=[
            jax.ShapeDtypeStruct(x.shape, x.dtype),
            jax.ShapeDtypeStruct(v.shape, v.dtype),
            jax.ShapeDtypeStruct(mask.shape, jnp.bool_),
        ],
        scratch_shapes=[
            pltpu.VMEM((_NBUF, _GROUP) + x.shape[1:], x.dtype),
            pltpu.VMEM(v.shape, v.dtype),
            pltpu.VMEM(mask.shape, mask.dtype),
            pltpu.SemaphoreType.DMA((_NBUF,)),
            pltpu.SemaphoreType.DMA((_NBUF,)),
            pltpu.SemaphoreType.DMA((2,)),
            pltpu.SemaphoreType.DMA((1,)),
        ],
    )(x, v, mask)


def kernel(x, v, mask=None, uu=None):
    if mask is None:
        mask = jnp.ones_like(x[:, :1])
    xo, vo, mo = _trim(x, v, mask)
    return (xo, vo, mo, uu)


# speedup vs baseline: 1.0142x; 1.0142x over previous
import jax
import jax.numpy as jnp
from jax.experimental import pallas as pl
from jax.experimental.pallas import tpu as pltpu

_GROUP = 2
_NCHUNK = 16 // _GROUP
_NBUF = _NCHUNK
_RAHEAD = 5


def _trim_kernel(x_hbm, v_hbm, m_hbm, xo_hbm, vo_hbm, mo_ref,
                 xbuf, vbuf, mbuf, rsem, wsem, vsem, msem):
    def src(ref, i):
        return ref.at[pl.ds(i * _GROUP, _GROUP)]

    def rd(i):
        return pltpu.make_async_copy(
            src(x_hbm, i), xbuf.at[i % _NBUF], rsem.at[i % _NBUF])

    def wr(i):
        return pltpu.make_async_copy(
            xbuf.at[i % _NBUF], src(xo_hbm, i), wsem.at[i % _NBUF])

    for i in range(_RAHEAD):
        rd(i).start()

    m_rd = pltpu.make_async_copy(m_hbm, mbuf, msem.at[0])
    m_rd.start()
    v_rd = pltpu.make_async_copy(v_hbm, vbuf, vsem.at[0])
    v_wr = pltpu.make_async_copy(vbuf, vo_hbm, vsem.at[1])
    v_rd.start()
    m_rd.wait()
    mo_ref[...] = mbuf[...] != 0.0
    v_rd.wait()
    v_wr.start()

    for i in range(_NCHUNK):
        rd(i).wait()
        wr(i).start()
        nxt = i + _RAHEAD
        if nxt < _NCHUNK:
            if nxt >= _NBUF:
                wr(nxt - _NBUF).wait()
            rd(nxt).start()
    for i in range(_NCHUNK - min(_NBUF, _NCHUNK), _NCHUNK):
        wr(i).wait()
    v_wr.wait()


def _trim(x, v, mask):
    hbm = pl.BlockSpec(memory_space=pltpu.MemorySpace.HBM)
    return pl.pallas_call(
        _trim_kernel,
        in_specs=[hbm, hbm, hbm],
        out_specs=[hbm, hbm,
                   pl.BlockSpec(memory_space=pltpu.MemorySpace.VMEM)],
        out_shape=[
            jax.ShapeDtypeStruct(x.shape, x.dtype),
            jax.ShapeDtypeStruct(v.shape, v.dtype),
            jax.ShapeDtypeStruct(mask.shape, jnp.bool_),
        ],
        scratch_shapes=[
            pltpu.VMEM((_NBUF, _GROUP) + x.shape[1:], x.dtype),
            pltpu.VMEM(v.shape, v.dtype),
            pltpu.VMEM(mask.shape, mask.dtype),
            pltpu.SemaphoreType.DMA((_NBUF,)),
            pltpu.SemaphoreType.DMA((_NBUF,)),
            pltpu.SemaphoreType.DMA((2,)),
            pltpu.SemaphoreType.DMA((1,)),
        ],
    )(x, v, mask)


def kernel(x, v, mask=None, uu=None):
    if mask is None:
        mask = jnp.ones_like(x[:, :1])
    xo, vo, mo = _trim(x, v, mask)
    return (xo, vo, mo, uu)
